# Initial kernel scaffold; baseline (speedup 1.0000x reference)
import jax
import jax.numpy as jnp
from jax.experimental import pallas as pl


def kernel(tables, token_ids, group_idx):
    raise NotImplementedError("write your pallas kernel here")


# speedup vs baseline: 7.1510x; 7.1510x over previous
import functools

import jax
import jax.numpy as jnp
from jax import lax
from jax.experimental import pallas as pl
from jax.experimental.pallas import tpu as pltpu
from jax.experimental.pallas import tpu_sc as plsc

_info = plsc.get_sparse_core_info()
_NC = _info.num_cores
_NS = _info.num_subcores
_L = _info.num_lanes
_NW = _NC * _NS


@functools.lru_cache(maxsize=None)
def _make_gather(B, D, chunk):
    bpw = B // _NW
    nchunk = bpw // chunk
    mesh = plsc.VectorSubcoreMesh(core_axis_name="c", subcore_axis_name="s")

    @functools.partial(
        pl.kernel,
        mesh=mesh,
        out_type=jax.ShapeDtypeStruct((B, D), jnp.float32),
        scratch_types=[
            pltpu.VMEM((bpw,), jnp.int32),
            pltpu.VMEM((_L,), jnp.int32),
            pltpu.VMEM((chunk, D), jnp.float32),
            pltpu.VMEM((chunk, D), jnp.float32),
            pltpu.SemaphoreType.DMA,
            pltpu.SemaphoreType.DMA,
        ],
    )
    def k(table_hbm, ids_hbm, gofs_hbm, out_hbm, idx_v, g_v, buf0, buf1,
          sem0, sem1):
        wid = lax.axis_index("s") * _NC + lax.axis_index("c")
        base = wid * bpw
        pltpu.sync_copy(ids_hbm.at[pl.ds(base, bpw)], idx_v)
        pltpu.sync_copy(gofs_hbm, g_v)
        g = g_v[...]
        for i in range(bpw // _L):
            sl = pl.ds(i * _L, _L)
            idx_v[sl] = idx_v[sl] + g

        bufs = (buf0, buf1)
        sems = (sem0, sem1)
        copies = [
            pltpu.async_copy(
                table_hbm.at[idx_v.at[pl.ds(0, chunk)]], bufs[0], sems[0])
        ]
        for c in range(nchunk):
            if c + 1 < nchunk:
                copies.append(pltpu.async_copy(
                    table_hbm.at[idx_v.at[pl.ds((c + 1) * chunk, chunk)]],
                    bufs[(c + 1) % 2], sems[(c + 1) % 2]))
            copies[c].wait()
            pltpu.sync_copy(bufs[c % 2],
                            out_hbm.at[pl.ds(base + c * chunk, chunk)])

    return k


def kernel(tables, token_ids, group_idx):
    G, V, D = tables.shape
    B, S = token_ids.shape
    table_flat = tables.reshape(G * V, D)
    ids = token_ids.reshape(B)
    gofs = jnp.broadcast_to(
        jnp.asarray(group_idx, jnp.int32) * jnp.int32(V), (_L,))
    out = _make_gather(B, D, 32)(table_flat, ids, gofs)
    return out.reshape(B, S, D)


# speedup vs baseline: 10.4629x; 1.4632x over previous
import functools

import jax
import jax.numpy as jnp
from jax import lax
from jax.experimental import pallas as pl
from jax.experimental.pallas import tpu as pltpu
from jax.experimental.pallas import tpu_sc as plsc

_info = plsc.get_sparse_core_info()
_NC = _info.num_cores
_NS = _info.num_subcores
_L = _info.num_lanes
_NW = _NC * _NS


@functools.lru_cache(maxsize=None)
def _make_gather(B, D, chunk):
    bpw = B // _NW
    nchunk = bpw // chunk
    mesh = plsc.VectorSubcoreMesh(core_axis_name="c", subcore_axis_name="s")

    @functools.partial(
        pl.kernel,
        mesh=mesh,
        out_type=jax.ShapeDtypeStruct((B, 1, D), jnp.float32),
        scratch_types=[
            pltpu.VMEM((bpw,), jnp.int32),
            pltpu.VMEM((_L,), jnp.int32),
            pltpu.VMEM((chunk, D), jnp.float32),
            pltpu.VMEM((chunk, D), jnp.float32),
            pltpu.SemaphoreType.DMA,
            pltpu.SemaphoreType.DMA,
        ],
    )
    def k(table_hbm, ids_hbm, gofs_hbm, out_hbm, idx_v, g_v, buf0, buf1,
          sem0, sem1):
        wid = lax.axis_index("s") * _NC + lax.axis_index("c")
        base = wid * bpw
        pltpu.sync_copy(ids_hbm.at[pl.ds(base, bpw)], idx_v)
        pltpu.sync_copy(gofs_hbm, g_v)
        g = g_v[...]
        for i in range(bpw // _L):
            sl = pl.ds(i * _L, _L)
            idx_v[sl] = idx_v[sl] + g

        bufs = (buf0, buf1)
        sems = (sem0, sem1)
        copies = [
            pltpu.async_copy(
                table_hbm.at[idx_v.at[pl.ds(0, chunk)]], bufs[0], sems[0])
        ]
        for c in range(nchunk):
            if c + 1 < nchunk:
                copies.append(pltpu.async_copy(
                    table_hbm.at[idx_v.at[pl.ds((c + 1) * chunk, chunk)]],
                    bufs[(c + 1) % 2], sems[(c + 1) % 2]))
            copies[c].wait()
            pltpu.sync_copy(bufs[c % 2],
                            out_hbm.at[pl.ds(base + c * chunk, chunk), 0])

    return k


def kernel(tables, token_ids, group_idx):
    G, V, D = tables.shape
    B, S = token_ids.shape
    table_flat = tables.reshape(G * V, D)
    ids = token_ids.reshape(B)
    gofs = jnp.broadcast_to(
        jnp.asarray(group_idx, jnp.int32) * jnp.int32(V), (_L,))
    out = _make_gather(B, D, 32)(table_flat, ids, gofs)
    return out.reshape(B, S, D) if S != 1 else out


# speedup vs baseline: 10.7892x; 1.0312x over previous
import functools

import jax
import jax.numpy as jnp
from jax import lax
from jax.experimental import pallas as pl
from jax.experimental.pallas import tpu as pltpu
from jax.experimental.pallas import tpu_sc as plsc

_info = plsc.get_sparse_core_info()
_NC = _info.num_cores
_NS = _info.num_subcores
_L = _info.num_lanes
_NW = _NC * _NS


@functools.lru_cache(maxsize=None)
def _make_gather(B, D, chunk):
    bpw = B // _NW
    sizes = []
    left = bpw
    while left > 0:
        sizes.append(min(chunk, left))
        left -= sizes[-1]
    offs = [sum(sizes[:i]) for i in range(len(sizes))]
    nchunk = len(sizes)
    mesh = plsc.VectorSubcoreMesh(core_axis_name="c", subcore_axis_name="s")

    @functools.partial(
        pl.kernel,
        mesh=mesh,
        out_type=jax.ShapeDtypeStruct((B, 1, D), jnp.float32),
        scratch_types=[
            pltpu.VMEM((bpw,), jnp.int32),
            pltpu.VMEM((_L,), jnp.int32),
            pltpu.VMEM((chunk, D), jnp.float32),
            pltpu.VMEM((chunk, D), jnp.float32),
            pltpu.SemaphoreType.DMA,
            pltpu.SemaphoreType.DMA,
        ],
    )
    def k(table_hbm, ids_hbm, gofs_hbm, out_hbm, idx_v, g_v, buf0, buf1,
          sem0, sem1):
        wid = lax.axis_index("s") * _NC + lax.axis_index("c")
        base = wid * bpw
        pltpu.sync_copy(ids_hbm.at[pl.ds(base, bpw)], idx_v)
        pltpu.sync_copy(gofs_hbm, g_v)
        g = g_v[...]
        for i in range(bpw // _L):
            sl = pl.ds(i * _L, _L)
            idx_v[sl] = idx_v[sl] + g

        bufs = (buf0, buf1)
        sems = (sem0, sem1)

        def gather(c):
            return pltpu.async_copy(
                table_hbm.at[idx_v.at[pl.ds(offs[c], sizes[c])]],
                bufs[c % 2].at[pl.ds(0, sizes[c])], sems[c % 2])

        copies = [gather(0)]
        for c in range(nchunk):
            if c + 1 < nchunk:
                copies.append(gather(c + 1))
            copies[c].wait()
            pltpu.sync_copy(bufs[c % 2].at[pl.ds(0, sizes[c])],
                            out_hbm.at[pl.ds(base + offs[c], sizes[c]), 0])

    return k


def kernel(tables, token_ids, group_idx):
    G, V, D = tables.shape
    B, S = token_ids.shape
    table_flat = tables.reshape(G * V, D)
    ids = token_ids.reshape(B)
    gofs = jnp.broadcast_to(
        jnp.asarray(group_idx, jnp.int32) * jnp.int32(V), (_L,))
    out = _make_gather(B, D, 48)(table_flat, ids, gofs)
    return out.reshape(B, S, D) if S != 1 else out


# speedup vs baseline: 11.0490x; 1.0241x over previous
import functools

import jax
import jax.numpy as jnp
from jax import lax
from jax.experimental import pallas as pl
from jax.experimental.pallas import tpu as pltpu
from jax.experimental.pallas import tpu_sc as plsc

_info = plsc.get_sparse_core_info()
_NC = _info.num_cores
_NS = _info.num_subcores
_L = _info.num_lanes
_NW = _NC * _NS


@functools.lru_cache(maxsize=None)
def _make_gather(B, V, D, chunk):
    bpw = B // _NW
    sizes = []
    left = bpw
    while left > 0:
        sizes.append(min(chunk, left))
        left -= sizes[-1]
    offs = [sum(sizes[:i]) for i in range(len(sizes))]
    n = len(sizes)
    mesh = plsc.VectorSubcoreMesh(core_axis_name="c", subcore_axis_name="s")

    @functools.partial(
        pl.kernel,
        mesh=mesh,
        out_type=jax.ShapeDtypeStruct((B, 1, D), jnp.float32),
        scratch_types=[
            pltpu.VMEM((bpw,), jnp.int32),
            pltpu.VMEM((_L,), jnp.int32),
            pltpu.VMEM((chunk, D), jnp.float32),
            pltpu.VMEM((chunk, D), jnp.float32),
            pltpu.SemaphoreType.DMA,
            pltpu.SemaphoreType.DMA,
            pltpu.SemaphoreType.DMA,
            pltpu.SemaphoreType.DMA,
        ],
    )
    def k(table_hbm, ids_hbm, gid_hbm, out_hbm, idx_v, g_v, buf0, buf1,
          sg0, sg1, sw0, sw1):
        wid = lax.axis_index("s") * _NC + lax.axis_index("c")
        base = wid * bpw
        pltpu.sync_copy(ids_hbm.at[pl.ds(base, bpw)], idx_v)
        g_v[...] = jnp.zeros((_L,), jnp.int32)
        pltpu.sync_copy(gid_hbm, g_v.at[pl.ds(0, 1)])
        gofs = lax.gather(
            g_v[...], jnp.zeros((_L, 1), jnp.int32),
            lax.GatherDimensionNumbers(offset_dims=(),
                                       collapsed_slice_dims=(0,),
                                       start_index_map=(0,)),
            (1,), mode=lax.GatherScatterMode.PROMISE_IN_BOUNDS) * V
        for i in range(bpw // _L):
            sl = pl.ds(i * _L, _L)
            idx_v[sl] = idx_v[sl] + gofs

        bufs = (buf0, buf1)
        gsems = (sg0, sg1)
        wsems = (sw0, sw1)

        def gather(c):
            return pltpu.async_copy(
                table_hbm.at[idx_v.at[pl.ds(offs[c], sizes[c])]],
                bufs[c % 2].at[pl.ds(0, sizes[c])], gsems[c % 2])

        def write(c):
            return pltpu.async_copy(
                bufs[c % 2].at[pl.ds(0, sizes[c])],
                out_hbm.at[pl.ds(base + offs[c], sizes[c]), 0],
                wsems[c % 2])

        gath = [None] * n
        wr = [None] * n
        gath[0] = gather(0)
        if n > 1:
            gath[1] = gather(1)
        drained = [False] * n
        for c in range(n):
            gath[c].wait()
            wr[c] = write(c)
            if c + 2 < n:
                wr[c].wait()
                drained[c] = True
                gath[c + 2] = gather(c + 2)
        for c in range(n):
            if not drained[c]:
                wr[c].wait()

    return k


def kernel(tables, token_ids, group_idx):
    G, V, D = tables.shape
    B, S = token_ids.shape
    table_flat = tables.reshape(G * V, D)
    ids = token_ids.reshape(B * S)
    gid = jnp.asarray(group_idx, jnp.int32).reshape(1)
    out = _make_gather(B * S, V, D, 48)(table_flat, ids, gid)
    return out.reshape(B, S, D) if S != 1 else out


# speedup vs baseline: 11.1254x; 1.0069x over previous
import functools

import jax
import jax.numpy as jnp
from jax import lax
from jax.experimental import pallas as pl
from jax.experimental.pallas import tpu as pltpu
from jax.experimental.pallas import tpu_sc as plsc

_info = plsc.get_sparse_core_info()
_NC = _info.num_cores
_NS = _info.num_subcores
_L = _info.num_lanes
_NW = _NC * _NS


@functools.lru_cache(maxsize=None)
def _make_gather(B, V, D, chunk):
    bpw = B // _NW
    sizes = []
    left = bpw
    while left > 0:
        sizes.append(min(chunk, left))
        left -= sizes[-1]
    offs = [sum(sizes[:i]) for i in range(len(sizes))]
    n = len(sizes)
    mesh = plsc.VectorSubcoreMesh(core_axis_name="c", subcore_axis_name="s")

    @functools.partial(
        pl.kernel,
        mesh=mesh,
        out_type=jax.ShapeDtypeStruct((B, 1, D), jnp.float32),
        scratch_types=[
            pltpu.VMEM((bpw,), jnp.int32),
            pltpu.VMEM((_L,), jnp.int32),
            pltpu.VMEM((chunk, D), jnp.float32),
            pltpu.VMEM((chunk, D), jnp.float32),
            pltpu.SemaphoreType.DMA,
            pltpu.SemaphoreType.DMA,
            pltpu.SemaphoreType.DMA,
            pltpu.SemaphoreType.DMA,
        ],
    )
    def k(table_hbm, ids_hbm, gid_hbm, out_hbm, idx_v, g_v, buf0, buf1,
          sg0, sg1, sw0, sw1):
        wid = lax.axis_index("s") * _NC + lax.axis_index("c")
        base = wid * bpw
        cp_ids = pltpu.async_copy(ids_hbm.at[pl.ds(base, bpw)], idx_v, sg0)
        cp_gid = pltpu.async_copy(gid_hbm, g_v.at[pl.ds(0, 1)], sg1)
        cp_gid.wait()
        gofs = lax.gather(
            g_v[...], jnp.zeros((_L, 1), jnp.int32),
            lax.GatherDimensionNumbers(offset_dims=(),
                                       collapsed_slice_dims=(0,),
                                       start_index_map=(0,)),
            (1,), mode=lax.GatherScatterMode.PROMISE_IN_BOUNDS) * V
        cp_ids.wait()

        bufs = (buf0, buf1)
        gsems = (sg0, sg1)
        wsems = (sw0, sw1)

        def add_offsets(c):
            for i in range(offs[c] // _L, (offs[c] + sizes[c]) // _L):
                sl = pl.ds(i * _L, _L)
                idx_v[sl] = idx_v[sl] + gofs

        def gather(c):
            return pltpu.async_copy(
                table_hbm.at[idx_v.at[pl.ds(offs[c], sizes[c])]],
                bufs[c % 2].at[pl.ds(0, sizes[c])], gsems[c % 2])

        def write(c):
            return pltpu.async_copy(
                bufs[c % 2].at[pl.ds(0, sizes[c])],
                out_hbm.at[pl.ds(base + offs[c], sizes[c]), 0],
                wsems[c % 2])

        gath = [None] * n
        wr = [None] * n
        add_offsets(0)
        gath[0] = gather(0)
        if n > 1:
            add_offsets(1)
            gath[1] = gather(1)
        for c in range(2, n):
            add_offsets(c)
        drained = [False] * n
        for c in range(n):
            gath[c].wait()
            wr[c] = write(c)
            if c + 2 < n:
                wr[c].wait()
                drained[c] = True
                gath[c + 2] = gather(c + 2)
        for c in range(n):
            if not drained[c]:
                wr[c].wait()

    return k


def kernel(tables, token_ids, group_idx):
    G, V, D = tables.shape
    B, S = token_ids.shape
    table_flat = tables.reshape(G * V, D)
    ids = token_ids.reshape(B * S)
    gid = jnp.asarray(group_idx, jnp.int32).reshape(1)
    out = _make_gather(B * S, V, D, 48)(table_flat, ids, gid)
    return out.reshape(B, S, D) if S != 1 else out


# speedup vs baseline: 11.2517x; 1.0114x over previous
import functools

import jax
import jax.numpy as jnp
from jax import lax
from jax.experimental import pallas as pl
from jax.experimental.pallas import tpu as pltpu
from jax.experimental.pallas import tpu_sc as plsc

_info = plsc.get_sparse_core_info()
_NC = _info.num_cores
_NS = _info.num_subcores
_L = _info.num_lanes
_NW = _NC * _NS


@functools.lru_cache(maxsize=None)
def _make_gather(B, V, D, chunk, nbuf):
    bpw = B // _NW
    sizes = []
    left = bpw
    while left > 0:
        sizes.append(min(chunk, left))
        left -= sizes[-1]
    offs = [sum(sizes[:i]) for i in range(len(sizes))]
    n = len(sizes)
    mesh = plsc.VectorSubcoreMesh(core_axis_name="c", subcore_axis_name="s")

    scratch = [
        pltpu.VMEM((bpw,), jnp.int32),
        pltpu.VMEM((_L,), jnp.int32),
    ]
    scratch += [pltpu.VMEM((chunk, D), jnp.float32) for _ in range(nbuf)]
    scratch += [pltpu.SemaphoreType.DMA for _ in range(2 * nbuf)]

    @functools.partial(
        pl.kernel,
        mesh=mesh,
        out_type=jax.ShapeDtypeStruct((B, 1, D), jnp.float32),
        scratch_types=scratch,
    )
    def k(table_hbm, ids_hbm, gid_hbm, out_hbm, idx_v, g_v, *bufs_sems):
        bufs = bufs_sems[:nbuf]
        gsems = bufs_sems[nbuf:2 * nbuf]
        wsems = bufs_sems[2 * nbuf:]
        wid = lax.axis_index("s") * _NC + lax.axis_index("c")
        base = wid * bpw
        cp_ids = pltpu.async_copy(ids_hbm.at[pl.ds(base, bpw)], idx_v,
                                  gsems[0])
        cp_gid = pltpu.async_copy(gid_hbm, g_v.at[pl.ds(0, 1)], gsems[1])
        cp_gid.wait()
        gofs = lax.gather(
            g_v[...], jnp.zeros((_L, 1), jnp.int32),
            lax.GatherDimensionNumbers(offset_dims=(),
                                       collapsed_slice_dims=(0,),
                                       start_index_map=(0,)),
            (1,), mode=lax.GatherScatterMode.PROMISE_IN_BOUNDS) * V
        cp_ids.wait()

        def add_offsets(c):
            for i in range(offs[c] // _L, (offs[c] + sizes[c]) // _L):
                sl = pl.ds(i * _L, _L)
                idx_v[sl] = idx_v[sl] + gofs

        def gather(c):
            return pltpu.async_copy(
                table_hbm.at[idx_v.at[pl.ds(offs[c], sizes[c])]],
                bufs[c % nbuf].at[pl.ds(0, sizes[c])], gsems[c % nbuf])

        def write(c):
            return pltpu.async_copy(
                bufs[c % nbuf].at[pl.ds(0, sizes[c])],
                out_hbm.at[pl.ds(base + offs[c], sizes[c]), 0],
                wsems[c % nbuf])

        gath = [None] * n
        wr = [None] * n
        head = min(nbuf, n)
        for c in range(head):
            add_offsets(c)
            gath[c] = gather(c)
        for c in range(head, n):
            add_offsets(c)
        drained = [False] * n
        for c in range(n):
            gath[c].wait()
            wr[c] = write(c)
            if c + nbuf < n:
                wr[c].wait()
                drained[c] = True
                gath[c + nbuf] = gather(c + nbuf)
        for c in range(n):
            if not drained[c]:
                wr[c].wait()

    return k


def kernel(tables, token_ids, group_idx):
    G, V, D = tables.shape
    B, S = token_ids.shape
    table_flat = tables.reshape(G * V, D)
    ids = token_ids.reshape(B * S)
    gid = jnp.asarray(group_idx, jnp.int32).reshape(1)
    out = _make_gather(B * S, V, D, 32, 3)(table_flat, ids, gid)
    return out.reshape(B, S, D) if S != 1 else out


# speedup vs baseline: 11.4863x; 1.0208x over previous
import functools

import jax
import jax.numpy as jnp
from jax import lax
from jax.experimental import pallas as pl
from jax.experimental.pallas import tpu as pltpu
from jax.experimental.pallas import tpu_sc as plsc

_info = plsc.get_sparse_core_info()
_NC = _info.num_cores
_NS = _info.num_subcores
_L = _info.num_lanes
_NW = _NC * _NS


@functools.lru_cache(maxsize=None)
def _make_gather(B, V, D, chunk, nbuf):
    bpw = B // _NW
    sizes = []
    left = bpw
    while left > 0:
        sizes.append(min(chunk, left))
        left -= sizes[-1]
    offs = [sum(sizes[:i]) for i in range(len(sizes))]
    n = len(sizes)
    mesh = plsc.VectorSubcoreMesh(core_axis_name="c", subcore_axis_name="s")

    scratch = [
        pltpu.VMEM((bpw,), jnp.int32),
        pltpu.VMEM((_L,), jnp.int32),
    ]
    scratch += [pltpu.VMEM((chunk, D), jnp.float32) for _ in range(nbuf)]
    scratch += [pltpu.SemaphoreType.DMA for _ in range(2 * nbuf)]

    @functools.partial(
        pl.kernel,
        mesh=mesh,
        out_type=jax.ShapeDtypeStruct((B, 1, D), jnp.float32),
        scratch_types=scratch,
    )
    def k(table_hbm, ids_hbm, gid_hbm, out_hbm, idx_v, g_v, *bufs_sems):
        bufs = bufs_sems[:nbuf]
        gsems = bufs_sems[nbuf:2 * nbuf]
        wsems = bufs_sems[2 * nbuf:]
        wid = lax.axis_index("s") * _NC + lax.axis_index("c")
        base = wid * bpw
        cp_ids = pltpu.async_copy(ids_hbm.at[pl.ds(base, bpw)], idx_v,
                                  gsems[0])
        cp_gid = pltpu.async_copy(gid_hbm, g_v.at[pl.ds(0, 1)], gsems[1])
        cp_gid.wait()
        gofs = lax.gather(
            g_v[...], jnp.zeros((_L, 1), jnp.int32),
            lax.GatherDimensionNumbers(offset_dims=(),
                                       collapsed_slice_dims=(0,),
                                       start_index_map=(0,)),
            (1,), mode=lax.GatherScatterMode.PROMISE_IN_BOUNDS) * V
        cp_ids.wait()

        def add_offsets(c):
            for i in range(offs[c] // _L, (offs[c] + sizes[c]) // _L):
                sl = pl.ds(i * _L, _L)
                idx_v[sl] = idx_v[sl] + gofs

        def gather(c):
            return pltpu.async_copy(
                table_hbm.at[idx_v.at[pl.ds(offs[c], sizes[c])]],
                bufs[c % nbuf].at[pl.ds(0, sizes[c])], gsems[c % nbuf])

        def write(c):
            return pltpu.async_copy(
                bufs[c % nbuf].at[pl.ds(0, sizes[c])],
                out_hbm.at[pl.ds(base + offs[c], sizes[c]), 0],
                wsems[c % nbuf])

        gath = [None] * n
        wr = [None] * n
        head = min(nbuf, n)
        for c in range(head):
            add_offsets(c)
            gath[c] = gather(c)
        for c in range(head, n):
            add_offsets(c)
        drained = [False] * n
        for c in range(n):
            gath[c].wait()
            wr[c] = write(c)
            if c + nbuf < n:
                wr[c].wait()
                drained[c] = True
                gath[c + nbuf] = gather(c + nbuf)
        for c in range(n):
            if not drained[c]:
                wr[c].wait()

    return k


def kernel(tables, token_ids, group_idx):
    G, V, D = tables.shape
    B, S = token_ids.shape
    table_flat = tables.reshape(G * V, D)
    ids = token_ids.reshape(B * S)
    gid = jnp.asarray(group_idx, jnp.int32).reshape(1)
    out = _make_gather(B * S, V, D, 16, 6)(table_flat, ids, gid)
    return out.reshape(B, S, D) if S != 1 else out
